# baseline (speedup 1.0000x reference)
import jax
import jax.numpy as jnp
from jax import lax
from jax.experimental import pallas as pl
from jax.experimental.pallas import tpu as pltpu
from jax.experimental.pallas import tpu_sc as plsc

B = 16384
VOCAB = 100000
EMB = 32
IDX_DIM = 64
HID = 256
U_FIELDS = 3
I_FIELDS = 4
N_FIELDS = U_FIELDS + I_FIELDS

GRP = 4
GS = 26624
LBLK = 2048
NBLK = GS // LBLK
MAXJ = (VOCAB - 1) // LBLK

NC = 2
NS = 16
NW = NC * NS
BW = B // NW
CHUNK = 128


def _tpack_body(x0, x1, x2, x3, o_ref):
    n = GRP * EMB
    eye = (lax.broadcasted_iota(jnp.int32, (n, n), 0)
           == lax.broadcasted_iota(jnp.int32, (n, n), 1)).astype(jnp.float32)
    x4 = jnp.concatenate([x[0] for x in (x0, x1, x2, x3)], axis=0)
    o_ref[...] = lax.dot_general(x4, eye,
                                 dimension_numbers=(((0,), (0,)), ((), ())),
                                 preferred_element_type=jnp.float32)


def _transpose_pack(tabT, n_fields):
    def spec(m):
        return pl.BlockSpec(
            (1, EMB, LBLK),
            lambda f, c: (f, 0, jnp.minimum(m * NBLK + c, MAXJ)))

    return pl.pallas_call(
        _tpack_body,
        grid=(n_fields, NBLK),
        in_specs=[spec(0), spec(1), spec(2), spec(3)],
        out_specs=pl.BlockSpec((LBLK, GRP * EMB), lambda f, c: (f * NBLK + c, 0)),
        out_shape=jax.ShapeDtypeStruct((n_fields * GS, GRP * EMB), jnp.float32),
    )(tabT, tabT, tabT, tabT)


def _sc_phase(tab_hbm, idx_hbm, out_hbm, f, base, idx_v, rows_v, sem):
    pltpu.sync_copy(idx_hbm.at[pl.ds(f * B + base, BW)], idx_v)
    copies = []
    for c in range(BW // CHUNK):
        copies.append(pltpu.async_copy(
            tab_hbm.at[idx_v.at[pl.ds(c * CHUNK, CHUNK)]],
            rows_v.at[pl.ds(c * CHUNK, CHUNK)],
            sem,
        ))
    for cp in copies:
        cp.wait()
    pltpu.sync_copy(rows_v, out_hbm.at[pl.ds(base, BW)])


def _sc_gather_u_body(u_tab, u_idx, o0, o1, o2, idx_v, rows_v, sem):
    base = (lax.axis_index("s") * NC + lax.axis_index("c")) * BW
    for f, out in enumerate((o0, o1, o2)):
        _sc_phase(u_tab, u_idx, out, f, base, idx_v, rows_v, sem)


def _sc_gather_i_body(i_tab, i_idx, o0, o1, o2, o3, idx_v, rows_v, sem):
    base = (lax.axis_index("s") * NC + lax.axis_index("c")) * BW
    for f, out in enumerate((o0, o1, o2, o3)):
        _sc_phase(i_tab, i_idx, out, f, base, idx_v, rows_v, sem)


def _mlp_body(idx_ref, g0, g1, g2, g3, g4, g5, g6, m_ref,
              w1_ref, w1p_bf_ref, b1_ref, w2_ref, b2_ref, o_ref):
    lane_grp = lax.broadcasted_iota(jnp.int32, (1, GRP * EMB), 1) // EMB
    h = jnp.dot(idx_ref[...].astype(jnp.bfloat16), w1_ref[...],
                preferred_element_type=jnp.float32)
    for f, g in enumerate((g0, g1, g2, g3, g4, g5, g6)):
        sel = jnp.where(m_ref[:, f:f + 1] == lane_grp, g[...], 0.0)
        lo = f * GRP * EMB
        h += jnp.dot(sel.astype(jnp.bfloat16),
                     w1p_bf_ref[lo:lo + GRP * EMB, :],
                     preferred_element_type=jnp.float32)
    h = jnp.maximum(h + b1_ref[...], 0.0)
    logits = jnp.dot(h.astype(jnp.bfloat16), w2_ref[...],
                     preferred_element_type=jnp.float32)
    logits += b2_ref[...]
    m = jnp.max(logits, axis=-1, keepdims=True)
    e = jnp.exp(logits - m)
    o_ref[...] = e / jnp.sum(e, axis=-1, keepdims=True)


_MLP_BLK = 4096


def kernel(indexEmb, userFeatures, itemFeatures, user_table, item_table,
           W1, b1, W2, b2):
    uT = jnp.transpose(user_table, (0, 2, 1))
    iT = jnp.transpose(item_table, (0, 2, 1))
    p_u = _transpose_pack(uT, U_FIELDS)
    p_i = _transpose_pack(iT, I_FIELDS)

    f_off_u = jnp.arange(U_FIELDS, dtype=jnp.int32)[:, None] * GS
    f_off_i = jnp.arange(I_FIELDS, dtype=jnp.int32)[:, None] * GS
    uF_t = jnp.transpose(userFeatures)
    iF_t = jnp.transpose(itemFeatures)
    u_idx = (uF_t % GS + f_off_u).reshape(U_FIELDS * B)
    i_idx = (iF_t % GS + f_off_i).reshape(I_FIELDS * B)
    grp = jnp.concatenate([userFeatures // GS, itemFeatures // GS], axis=1)

    w1_fields = W1[IDX_DIM:].reshape(N_FIELDS, EMB, HID)
    w1_tiled = jnp.tile(w1_fields[:, None], (1, GRP, 1, 1))
    W1p = w1_tiled.reshape(N_FIELDS * GRP * EMB, HID).astype(jnp.bfloat16)

    mesh = plsc.VectorSubcoreMesh(core_axis_name="c", subcore_axis_name="s")

    def make_gather(body, nf):
        return pl.kernel(
            body,
            out_type=tuple(
                jax.ShapeDtypeStruct((B, GRP * EMB), jnp.float32)
                for _ in range(nf)
            ),
            mesh=mesh,
            scratch_types=[
                pltpu.VMEM((BW,), jnp.int32),
                pltpu.VMEM((BW, GRP * EMB), jnp.float32),
                pltpu.SemaphoreType.DMA,
            ],
            compiler_params=pltpu.CompilerParams(use_tc_tiling_on_sc=True),
        )

    fields_u = make_gather(_sc_gather_u_body, U_FIELDS)(p_u, u_idx)
    fields_i = make_gather(_sc_gather_i_body, I_FIELDS)(p_i, i_idx)
    fields = tuple(fields_u) + tuple(fields_i)

    grid = (B // _MLP_BLK,)
    row_spec = pl.BlockSpec((_MLP_BLK, GRP * EMB), lambda i: (i, 0))
    out = pl.pallas_call(
        _mlp_body,
        grid=grid,
        in_specs=[
            pl.BlockSpec((_MLP_BLK, IDX_DIM), lambda i: (i, 0)),
            row_spec, row_spec, row_spec,
            row_spec, row_spec, row_spec, row_spec,
            pl.BlockSpec((_MLP_BLK, N_FIELDS), lambda i: (i, 0)),
            pl.BlockSpec((IDX_DIM, HID), lambda i: (0, 0)),
            pl.BlockSpec((N_FIELDS * GRP * EMB, HID), lambda i: (0, 0)),
            pl.BlockSpec((1, HID), lambda i: (0, 0)),
            pl.BlockSpec((HID, 2), lambda i: (0, 0)),
            pl.BlockSpec((1, 2), lambda i: (0, 0)),
        ],
        out_specs=pl.BlockSpec((_MLP_BLK, 2), lambda i: (i, 0)),
        out_shape=jax.ShapeDtypeStruct((B, 2), jnp.float32),
    )(indexEmb, *fields, grp, W1[:IDX_DIM].astype(jnp.bfloat16), W1p,
      b1.reshape(1, HID), W2.astype(jnp.bfloat16), b2.reshape(1, 2))
    return out


# speedup vs baseline: 1.0092x; 1.0092x over previous
import jax
import jax.numpy as jnp
from jax import lax
from jax.experimental import pallas as pl
from jax.experimental.pallas import tpu as pltpu
from jax.experimental.pallas import tpu_sc as plsc

B = 16384
VOCAB = 100000
EMB = 32
IDX_DIM = 64
HID = 256
U_FIELDS = 3
I_FIELDS = 4
N_FIELDS = U_FIELDS + I_FIELDS

GRP = 4
GS = 26624
LBLK = 2048
NBLK = GS // LBLK
MAXJ = (VOCAB - 1) // LBLK

NC = 2
NS = 16
NW = NC * NS
BW = B // NW
CHUNK = 128


def _tpack_body(x0, x1, x2, x3, o_ref):
    n = GRP * EMB
    eye = (lax.broadcasted_iota(jnp.int32, (n, n), 0)
           == lax.broadcasted_iota(jnp.int32, (n, n), 1)).astype(jnp.float32)
    x4 = jnp.concatenate([x[0] for x in (x0, x1, x2, x3)], axis=0)
    o_ref[...] = lax.dot_general(x4, eye,
                                 dimension_numbers=(((0,), (0,)), ((), ())),
                                 preferred_element_type=jnp.float32)


def _transpose_pack(tabT, n_fields):
    def spec(m):
        return pl.BlockSpec(
            (1, EMB, LBLK),
            lambda f, c: (f, 0, jnp.minimum(m * NBLK + c, MAXJ)))

    return pl.pallas_call(
        _tpack_body,
        grid=(n_fields, NBLK),
        in_specs=[spec(0), spec(1), spec(2), spec(3)],
        out_specs=pl.BlockSpec((LBLK, GRP * EMB), lambda f, c: (f * NBLK + c, 0)),
        out_shape=jax.ShapeDtypeStruct((n_fields * GS, GRP * EMB), jnp.float32),
    )(tabT, tabT, tabT, tabT)


def _sc_phase(tab_hbm, idx_hbm, out_hbm, f, base, idx_v, rows_v, sem):
    pltpu.sync_copy(idx_hbm.at[pl.ds(f * B + base, BW)], idx_v)
    copies = []
    for c in range(BW // CHUNK):
        copies.append(pltpu.async_copy(
            tab_hbm.at[idx_v.at[pl.ds(c * CHUNK, CHUNK)]],
            rows_v.at[pl.ds(c * CHUNK, CHUNK)],
            sem,
        ))
    for cp in copies:
        cp.wait()
    pltpu.sync_copy(rows_v, out_hbm.at[pl.ds(base, BW)])


def _sc_gather_u_body(u_tab, u_idx, o0, o1, o2, idx_v, rows_v, sem):
    base = (lax.axis_index("s") * NC + lax.axis_index("c")) * BW
    for f, out in enumerate((o0, o1, o2)):
        _sc_phase(u_tab, u_idx, out, f, base, idx_v, rows_v, sem)


def _sc_gather_i_body(i_tab, i_idx, o0, o1, o2, o3, idx_v, rows_v, sem):
    base = (lax.axis_index("s") * NC + lax.axis_index("c")) * BW
    for f, out in enumerate((o0, o1, o2, o3)):
        _sc_phase(i_tab, i_idx, out, f, base, idx_v, rows_v, sem)


def _mlp_body(idx_ref, g0, g1, g2, g3, g4, g5, g6, m_ref,
              w1_ref, w1p_bf_ref, b1_ref, w2_ref, b2_ref, o_ref):
    lane_grp = lax.broadcasted_iota(jnp.int32, (1, GRP * EMB), 1) // EMB
    h = jnp.dot(idx_ref[...].astype(jnp.bfloat16), w1_ref[...],
                preferred_element_type=jnp.float32)
    for f, g in enumerate((g0, g1, g2, g3, g4, g5, g6)):
        sel = jnp.where(m_ref[:, f:f + 1] == lane_grp, g[...], 0.0)
        lo = f * GRP * EMB
        h += jnp.dot(sel.astype(jnp.bfloat16),
                     w1p_bf_ref[lo:lo + GRP * EMB, :],
                     preferred_element_type=jnp.float32)
    h = jnp.maximum(h + b1_ref[...], 0.0)
    logits = jnp.dot(h.astype(jnp.bfloat16), w2_ref[...],
                     preferred_element_type=jnp.float32)
    logits += b2_ref[...]
    m = jnp.max(logits, axis=-1, keepdims=True)
    e = jnp.exp(logits - m)
    o_ref[...] = e / jnp.sum(e, axis=-1, keepdims=True)


_MLP_BLK = 2048


def kernel(indexEmb, userFeatures, itemFeatures, user_table, item_table,
           W1, b1, W2, b2):
    uT = jnp.transpose(user_table, (0, 2, 1))
    iT = jnp.transpose(item_table, (0, 2, 1))
    p_u = _transpose_pack(uT, U_FIELDS)
    p_i = _transpose_pack(iT, I_FIELDS)

    f_off_u = jnp.arange(U_FIELDS, dtype=jnp.int32)[:, None] * GS
    f_off_i = jnp.arange(I_FIELDS, dtype=jnp.int32)[:, None] * GS
    uF_t = jnp.transpose(userFeatures)
    iF_t = jnp.transpose(itemFeatures)
    u_idx = (uF_t % GS + f_off_u).reshape(U_FIELDS * B)
    i_idx = (iF_t % GS + f_off_i).reshape(I_FIELDS * B)
    grp = jnp.concatenate([userFeatures // GS, itemFeatures // GS], axis=1)

    w1_fields = W1[IDX_DIM:].reshape(N_FIELDS, EMB, HID)
    w1_tiled = jnp.tile(w1_fields[:, None], (1, GRP, 1, 1))
    W1p = w1_tiled.reshape(N_FIELDS * GRP * EMB, HID).astype(jnp.bfloat16)

    mesh = plsc.VectorSubcoreMesh(core_axis_name="c", subcore_axis_name="s")

    def make_gather(body, nf):
        return pl.kernel(
            body,
            out_type=tuple(
                jax.ShapeDtypeStruct((B, GRP * EMB), jnp.float32)
                for _ in range(nf)
            ),
            mesh=mesh,
            scratch_types=[
                pltpu.VMEM((BW,), jnp.int32),
                pltpu.VMEM((BW, GRP * EMB), jnp.float32),
                pltpu.SemaphoreType.DMA,
            ],
            compiler_params=pltpu.CompilerParams(use_tc_tiling_on_sc=True),
        )

    fields_u = make_gather(_sc_gather_u_body, U_FIELDS)(p_u, u_idx)
    fields_i = make_gather(_sc_gather_i_body, I_FIELDS)(p_i, i_idx)
    fields = tuple(fields_u) + tuple(fields_i)

    grid = (B // _MLP_BLK,)
    row_spec = pl.BlockSpec((_MLP_BLK, GRP * EMB), lambda i: (i, 0))
    out = pl.pallas_call(
        _mlp_body,
        grid=grid,
        in_specs=[
            pl.BlockSpec((_MLP_BLK, IDX_DIM), lambda i: (i, 0)),
            row_spec, row_spec, row_spec,
            row_spec, row_spec, row_spec, row_spec,
            pl.BlockSpec((_MLP_BLK, N_FIELDS), lambda i: (i, 0)),
            pl.BlockSpec((IDX_DIM, HID), lambda i: (0, 0)),
            pl.BlockSpec((N_FIELDS * GRP * EMB, HID), lambda i: (0, 0)),
            pl.BlockSpec((1, HID), lambda i: (0, 0)),
            pl.BlockSpec((HID, 2), lambda i: (0, 0)),
            pl.BlockSpec((1, 2), lambda i: (0, 0)),
        ],
        out_specs=pl.BlockSpec((_MLP_BLK, 2), lambda i: (i, 0)),
        out_shape=jax.ShapeDtypeStruct((B, 2), jnp.float32),
    )(indexEmb, *fields, grp, W1[:IDX_DIM].astype(jnp.bfloat16), W1p,
      b1.reshape(1, HID), W2.astype(jnp.bfloat16), b2.reshape(1, 2))
    return out
